# Initial kernel scaffold; baseline (speedup 1.0000x reference)
import jax
import jax.numpy as jnp
from jax.experimental import pallas as pl


def kernel(R, t, p_CB, x, z, mask, neighbors, Wq, Wk, Wv, Wpair, spatial_coef, Wout, bout, ln_gamma, ln_beta):
    raise NotImplementedError("write your pallas kernel here")


# speedup vs baseline: 1.5716x; 1.5716x over previous
import functools

import jax
import jax.numpy as jnp
import numpy as np
from jax import lax
from jax.experimental import pallas as pl
from jax.experimental.pallas import tpu as pltpu
from jax.experimental.pallas import tpu_sc as plsc

B, L, M, F, C, H, QK, VD = 2, 384, 32, 128, 64, 12, 16, 16
N = B * L * M
NC, NS = 2, 16
NW = NC * NS
PER_W = N // NW
CHUNK = 128
NCH = PER_W // CHUNK

TQ = 8
GRID = (B * L) // TQ


def _sc_gather_body(x_hbm, z_hbm, ix_hbm, iz_hbm, xk_out, zz_out,
                    ixv, izv, xbuf, zbuf, s1, s2):
    wid = lax.axis_index("s") * NC + lax.axis_index("c")
    base = wid * PER_W

    def chunk(ci, carry):
        off = base + ci * CHUNK
        pltpu.sync_copy(ix_hbm.at[pl.ds(off, CHUNK)], ixv)
        pltpu.sync_copy(iz_hbm.at[pl.ds(off, CHUNK)], izv)
        a = pltpu.async_copy(x_hbm.at[ixv], xbuf, s1)
        b = pltpu.async_copy(z_hbm.at[izv], zbuf, s2)
        a.wait()
        b.wait()
        pltpu.sync_copy(xbuf, xk_out.at[pl.ds(off, CHUNK)])
        pltpu.sync_copy(zbuf, zz_out.at[pl.ds(off, CHUNK)])
        return carry

    lax.fori_loop(0, NCH, chunk, 0)


@functools.cache
def _make_sc_gather():
    return pl.kernel(
        _sc_gather_body,
        out_type=[
            jax.ShapeDtypeStruct((N, F), jnp.float32),
            jax.ShapeDtypeStruct((N, 2 * C), jnp.float32),
        ],
        mesh=plsc.VectorSubcoreMesh(
            core_axis_name="c", subcore_axis_name="s",
            num_cores=NC, num_subcores=NS),
        scratch_types=[
            pltpu.VMEM((CHUNK,), jnp.int32),
            pltpu.VMEM((CHUNK,), jnp.int32),
            pltpu.VMEM((CHUNK, F), jnp.float32),
            pltpu.VMEM((CHUNK, 2 * C), jnp.float32),
            pltpu.SemaphoreType.DMA,
            pltpu.SemaphoreType.DMA,
        ],
    )


def _tc_body(x_ref, xk_ref, zz_ref, idx_ref, p4_ref, geo_ref,
             wq_ref, wk_ref, wv_ref, wp_ref,
             w1_ref, w2_ref, w3_ref, bout_ref, coef_ref, lng_ref, lnb_ref,
             o_ref):
    x = x_ref[...]
    xk = xk_ref[...]
    zz = zz_ref[...]
    idxi = idx_ref[...]
    geo = geo_ref[...]

    q = jnp.dot(x, wq_ref[...], preferred_element_type=jnp.float32)
    k = jnp.dot(xk, wk_ref[...], preferred_element_type=jnp.float32)
    v = jnp.dot(xk, wv_ref[...], preferred_element_type=jnp.float32)

    par = (idxi & 1) > 0
    zk2 = jnp.where(par, zz[:, C:], zz[:, :C])
    zk = zk2.reshape(TQ, M, C)

    rows = jax.lax.broadcasted_iota(jnp.int32, (TQ * M, B * L), 1)
    sel = jnp.where(rows == idxi, 1.0, 0.0)
    pk4 = jnp.dot(sel, p4_ref[...], preferred_element_type=jnp.float32)
    pk = pk4.reshape(TQ, M, 4)[:, :, :3]

    ln = jnp.sum(k.reshape(TQ, M, H, QK) * q.reshape(TQ, 1, H, QK), axis=-1)
    lp = jnp.dot(zk2, wp_ref[...], preferred_element_type=jnp.float32)
    lp = lp.reshape(TQ, M, H)

    diff = pk - geo[:, 12:15].reshape(TQ, 1, 3)
    d2 = jnp.sum(diff * diff, axis=-1)
    gamma = jnp.log(1.0 + jnp.exp(coef_ref[...]))
    ls = d2[:, :, None] * (gamma.reshape(1, 1, H) *
                           (-np.sqrt(2.0 / 9.0) / 2.0))

    logits = (ln + lp + ls) * np.sqrt(1.0 / 3.0)
    mx = jnp.max(logits, axis=1, keepdims=True)
    e = jnp.exp(logits - mx)
    alpha = e / jnp.sum(e, axis=1, keepdims=True)
    alpha = alpha * geo[:, 15].reshape(TQ, 1, 1)

    a4 = alpha[:, :, :, None]
    fp2n = jnp.sum(a4 * zk.reshape(TQ, M, 1, C), axis=1)
    fnode = jnp.sum(a4 * v.reshape(TQ, M, H, QK), axis=1)
    aggr = jnp.sum(a4 * pk.reshape(TQ, M, 1, 3), axis=1)

    am = aggr - geo[:, 9:12].reshape(TQ, 1, 3)
    rmat = geo[:, :9].reshape(TQ, 1, 3, 3)
    fpt = jnp.sum(rmat * am[:, :, :, None], axis=2)
    dist = jnp.sqrt(jnp.sum(fpt * fpt, axis=-1))
    dirv = fpt / (dist[:, :, None] + 1e-4)
    small = jnp.concatenate(
        [fpt.reshape(TQ, 3 * H), dist, dirv.reshape(TQ, 3 * H),
         jnp.zeros((TQ, 12), jnp.float32)], axis=-1)

    acc = (jnp.dot(fp2n.reshape(TQ, H * C), w1_ref[...],
                   preferred_element_type=jnp.float32)
           + jnp.dot(fnode.reshape(TQ, H * QK), w2_ref[...],
                     preferred_element_type=jnp.float32)
           + jnp.dot(small, w3_ref[...],
                     preferred_element_type=jnp.float32)
           + bout_ref[...])
    feat_all = acc * geo[:, 15].reshape(TQ, 1)

    y = x + feat_all
    mu = jnp.mean(y, axis=-1, keepdims=True)
    yc = y - mu
    var = jnp.mean(yc * yc, axis=-1, keepdims=True)
    o_ref[...] = (yc * lax.rsqrt(var + 1e-5) * lng_ref[...]
                  + lnb_ref[...])


def _row_spec(rows, cols):
    return pl.BlockSpec((rows, cols), lambda i: (i, 0))


def _full_spec(rows, cols):
    return pl.BlockSpec((rows, cols), lambda i: (0, 0))


_TC_IN_SPECS = [
    _row_spec(TQ, F),
    _row_spec(TQ * M, F),
    _row_spec(TQ * M, 2 * C),
    _row_spec(TQ * M, 1),
    _full_spec(B * L, 4),
    _row_spec(TQ, 16),
    _full_spec(F, H * QK),
    _full_spec(F, H * QK),
    _full_spec(F, H * VD),
    _full_spec(C, H),
    _full_spec(H * C, F),
    _full_spec(H * QK, F),
    _full_spec(96, F),
    _full_spec(1, F),
    _full_spec(1, H),
    _full_spec(1, F),
    _full_spec(1, F),
]

_tc_attention = pl.pallas_call(
    _tc_body,
    grid=(GRID,),
    in_specs=_TC_IN_SPECS,
    out_specs=_row_spec(TQ, F),
    out_shape=jax.ShapeDtypeStruct((B * L, F), jnp.float32),
)


def kernel(R, t, p_CB, x, z, mask, neighbors, Wq, Wk, Wv, Wpair,
           spatial_coef, Wout, bout, ln_gamma, ln_beta):
    f32 = jnp.float32
    nb = neighbors.astype(jnp.int32)
    boff = (jnp.arange(B, dtype=jnp.int32) * L)[:, None, None]
    idx_x = (nb + boff).reshape(-1)
    qoff = (jnp.arange(B * L, dtype=jnp.int32) * L).reshape(B, L, 1)
    idx_z2 = ((nb + qoff) >> 1).reshape(-1)

    xf = x.reshape(B * L, F).astype(f32)
    z2_tab = z.reshape(B * L * L // 2, 2 * C).astype(f32)

    p4 = jnp.concatenate(
        [p_CB.reshape(B * L, 3).astype(f32), jnp.zeros((B * L, 1), f32)],
        axis=1)
    geo = jnp.concatenate(
        [R.reshape(B * L, 9).astype(f32), t.reshape(B * L, 3).astype(f32),
         p_CB.reshape(B * L, 3).astype(f32),
         mask.reshape(B * L, 1).astype(f32)], axis=1)

    xk_knn, zz_knn = _make_sc_gather()(xf, z2_tab, idx_x, idx_z2)

    idxf = idx_x.reshape(N, 1)

    w1 = Wout[:, :H * C].T
    w2 = Wout[:, H * C:H * C + H * QK].T
    w3 = jnp.concatenate(
        [Wout[:, H * C + H * QK:].T, jnp.zeros((12, F), f32)], axis=0)

    out = _tc_attention(
        xf, xk_knn, zz_knn, idxf, p4, geo,
        Wq.T, Wk.T, Wv.T, Wpair.T,
        w1, w2, w3,
        bout.reshape(1, F), spatial_coef.reshape(1, H),
        ln_gamma.reshape(1, F), ln_beta.reshape(1, F))
    return out.reshape(B, L, F)


# speedup vs baseline: 3.8778x; 2.4674x over previous
import functools

import jax
import jax.numpy as jnp
import numpy as np
from jax import lax
from jax.experimental import pallas as pl
from jax.experimental.pallas import tpu as pltpu
from jax.experimental.pallas import tpu_sc as plsc

B, L, M, F, C, H, QK, VD = 2, 384, 32, 128, 64, 12, 16, 16
N = B * L * M
NC, NS = 2, 16
NW = NC * NS
PER_W = N // NW
CHUNK = 128
NCH = PER_W // CHUNK

TQ = 64
TR = TQ * M
GRID = (B * L) // TQ


def _sc_gather_body(x_hbm, z_hbm, p_hbm, ix_hbm, iz_hbm,
                    xk_out, zz_out, pk_out,
                    ixv, izv, xbuf, zbuf, ptab, pkbuf, sx, sz):
    wid = lax.axis_index("s") * NC + lax.axis_index("c")
    base = wid * PER_W
    ia = pltpu.async_copy(ix_hbm.at[pl.ds(base, PER_W)], ixv, sx.at[0])
    ib = pltpu.async_copy(iz_hbm.at[pl.ds(base, PER_W)], izv, sz.at[0])
    pltpu.sync_copy(p_hbm, ptab)
    ia.wait()
    ib.wait()

    def issue(ci, slot):
        a = pltpu.async_copy(
            x_hbm.at[ixv.at[pl.ds(ci * CHUNK, CHUNK)]], xbuf.at[slot], sx.at[slot])
        b = pltpu.async_copy(
            z_hbm.at[izv.at[pl.ds(ci * CHUNK, CHUNK)]], zbuf.at[slot], sz.at[slot])
        return a, b

    pend = issue(0, 0)
    for ci in range(NCH):
        slot = ci % 2
        off = base + ci * CHUNK
        nxt = issue(ci + 1, 1 - slot) if ci + 1 < NCH else None

        def ploop(i2, c2, ci=ci):
            o = ci * (CHUNK * 4) + i2 * 16 + lax.iota(jnp.int32, 16)
            iv = plsc.load_gather(ixv, [o >> 2])
            vals = plsc.load_gather(ptab, [iv * 4 + (o & 3)])
            pkbuf[pl.ds(i2 * 16, 16)] = vals
            return c2

        lax.fori_loop(0, CHUNK * 4 // 16, ploop, 0)
        pltpu.sync_copy(pkbuf, pk_out.at[pl.ds(off * 4, CHUNK * 4)])
        a, b = pend
        a.wait()
        b.wait()
        pltpu.sync_copy(xbuf.at[slot], xk_out.at[pl.ds(off, CHUNK)])
        pltpu.sync_copy(zbuf.at[slot], zz_out.at[pl.ds(off, CHUNK)])
        pend = nxt


@functools.cache
def _make_sc_gather():
    return pl.kernel(
        _sc_gather_body,
        out_type=[
            jax.ShapeDtypeStruct((N, F), jnp.float32),
            jax.ShapeDtypeStruct((N, 2 * C), jnp.float32),
            jax.ShapeDtypeStruct((N * 4,), jnp.float32),
        ],
        mesh=plsc.VectorSubcoreMesh(
            core_axis_name="c", subcore_axis_name="s",
            num_cores=NC, num_subcores=NS),
        compiler_params=pltpu.CompilerParams(needs_layout_passes=False),
        scratch_types=[
            pltpu.VMEM((PER_W,), jnp.int32),
            pltpu.VMEM((PER_W,), jnp.int32),
            pltpu.VMEM((2, CHUNK, F), jnp.float32),
            pltpu.VMEM((2, CHUNK, 2 * C), jnp.float32),
            pltpu.VMEM((B * L * 4,), jnp.float32),
            pltpu.VMEM((CHUNK * 4,), jnp.float32),
            pltpu.SemaphoreType.DMA((2,)),
            pltpu.SemaphoreType.DMA((2,)),
        ],
    )


def _tc_body(x_ref, xk_ref, zz_ref, pk_ref, idx_ref, geo_ref,
             wq_ref, wk_ref, wv_ref, wp_ref,
             seg_ref, segt_ref, seg768_ref, tile12_ref, s8_ref,
             w1_ref, w2_ref, w3_ref, bout_ref, coef_ref, lng_ref, lnb_ref,
             o_ref):
    x = x_ref[...]
    xk = xk_ref[...]
    zz = zz_ref[...]
    pk4 = pk_ref[...]
    idxi = idx_ref[...]
    geo = geo_ref[...]

    q = jnp.dot(x, wq_ref[...], preferred_element_type=jnp.float32)
    k = jnp.dot(xk, wk_ref[...], preferred_element_type=jnp.float32)
    v = jnp.dot(xk, wv_ref[...], preferred_element_type=jnp.float32)

    par = (idxi & 1) > 0
    zk2 = jnp.where(par, zz[:, C:], zz[:, :C])

    qb = jnp.broadcast_to(q.reshape(TQ, 1, H * QK),
                          (TQ, M, H * QK)).reshape(TR, H * QK)
    ln2 = jnp.dot(k * qb, seg_ref[...], preferred_element_type=jnp.float32)

    lp2 = jnp.dot(zk2, wp_ref[...], preferred_element_type=jnp.float32)

    pselfb = jnp.broadcast_to(geo[:, 0:4].reshape(TQ, 1, 4),
                              (TQ, M, 4)).reshape(TR, 4)
    diff = pk4 - pselfb
    d2 = jnp.sum(diff * diff, axis=-1, keepdims=True)
    gamma = jnp.log(1.0 + jnp.exp(coef_ref[...]))
    ls2 = d2 * (gamma * (-np.sqrt(2.0 / 9.0) / 2.0))

    logits = (ln2 + lp2 + ls2) * np.sqrt(1.0 / 3.0)
    lg3 = logits.reshape(TQ, M, H)
    mx = jnp.max(lg3, axis=1, keepdims=True)
    e = jnp.exp(lg3 - mx)
    alpha3 = e / jnp.sum(e, axis=1, keepdims=True)
    alpha = alpha3.reshape(TR, H)

    s8 = s8_ref[...]
    av = v * jnp.dot(alpha, segt_ref[...], preferred_element_type=jnp.float32)
    fnode = jnp.dot(s8, av, preferred_element_type=jnp.float32)

    zt = jnp.dot(zk2, tile12_ref[...], preferred_element_type=jnp.float32)
    azt = zt * jnp.dot(alpha, seg768_ref[...], preferred_element_type=jnp.float32)
    fp2n = jnp.dot(s8, azt, preferred_element_type=jnp.float32)

    ag = [jnp.dot(s8, alpha * pk4[:, j:j + 1],
                  preferred_element_type=jnp.float32) for j in range(3)]
    am = [ag[j] - geo[:, 4 + j:5 + j] for j in range(3)]
    fpt = [am[0] * geo[:, 8 + i:9 + i] + am[1] * geo[:, 11 + i:12 + i]
           + am[2] * geo[:, 14 + i:15 + i] for i in range(3)]
    dist = jnp.sqrt(fpt[0] * fpt[0] + fpt[1] * fpt[1] + fpt[2] * fpt[2])
    inv = 1.0 / (dist + 1e-4)
    small = jnp.concatenate(
        [fpt[0], fpt[1], fpt[2], dist, fpt[0] * inv, fpt[1] * inv,
         fpt[2] * inv, jnp.zeros((TQ, H), jnp.float32)], axis=-1)

    acc = (jnp.dot(fp2n, w1_ref[...], preferred_element_type=jnp.float32)
           + jnp.dot(fnode, w2_ref[...], preferred_element_type=jnp.float32)
           + jnp.dot(small, w3_ref[...], preferred_element_type=jnp.float32)
           + bout_ref[...])
    feat_all = acc * geo[:, 17:18]

    y = x + feat_all
    mu = jnp.mean(y, axis=-1, keepdims=True)
    yc = y - mu
    var = jnp.mean(yc * yc, axis=-1, keepdims=True)
    o_ref[...] = (yc * lax.rsqrt(var + 1e-5) * lng_ref[...]
                  + lnb_ref[...])


def _row_spec(rows, cols):
    return pl.BlockSpec((rows, cols), lambda i: (i, 0))


def _full_spec(rows, cols):
    return pl.BlockSpec((rows, cols), lambda i: (0, 0))


_TC_IN_SPECS = [
    _row_spec(TQ, F),
    _row_spec(TR, F),
    _row_spec(TR, 2 * C),
    _row_spec(TR, 4),
    _row_spec(TR, 1),
    _row_spec(TQ, 24),
    _full_spec(F, H * QK),
    _full_spec(F, H * QK),
    _full_spec(F, H * VD),
    _full_spec(C, H),
    _full_spec(H * QK, H),
    _full_spec(H, H * QK),
    _full_spec(H, H * C),
    _full_spec(C, H * C),
    _full_spec(TQ, TR),
    _full_spec(H * C, F),
    _full_spec(H * QK, F),
    _full_spec(96, F),
    _full_spec(1, F),
    _full_spec(1, H),
    _full_spec(1, F),
    _full_spec(1, F),
]

_tc_attention = pl.pallas_call(
    _tc_body,
    grid=(GRID,),
    in_specs=_TC_IN_SPECS,
    out_specs=_row_spec(TQ, F),
    out_shape=jax.ShapeDtypeStruct((B * L, F), jnp.float32),
)

_SEG = np.kron(np.eye(H), np.ones((QK, 1))).astype(np.float32)
_SEGT = np.kron(np.eye(H), np.ones((1, QK))).astype(np.float32)
_SEG768 = np.kron(np.eye(H), np.ones((1, C))).astype(np.float32)
_TILE12 = np.tile(np.eye(C), (1, H)).astype(np.float32)
_S8 = np.kron(np.eye(TQ), np.ones((1, M))).astype(np.float32)


def kernel(R, t, p_CB, x, z, mask, neighbors, Wq, Wk, Wv, Wpair,
           spatial_coef, Wout, bout, ln_gamma, ln_beta):
    f32 = jnp.float32
    nb = neighbors.astype(jnp.int32)
    boff = (jnp.arange(B, dtype=jnp.int32) * L)[:, None, None]
    idx_x = (nb + boff).reshape(-1)
    qoff = (jnp.arange(B * L, dtype=jnp.int32) * L).reshape(B, L, 1)
    idx_z2 = ((nb + qoff) >> 1).reshape(-1)

    xf = x.reshape(B * L, F).astype(f32)
    z2_tab = z.reshape(B * L * L // 2, 2 * C).astype(f32)
    p4 = jnp.concatenate(
        [p_CB.reshape(B * L, 3).astype(f32), jnp.zeros((B * L, 1), f32)],
        axis=1).reshape(-1)

    geo = jnp.concatenate(
        [p_CB.reshape(B * L, 3).astype(f32), jnp.zeros((B * L, 1), f32),
         t.reshape(B * L, 3).astype(f32), jnp.zeros((B * L, 1), f32),
         R.reshape(B * L, 9).astype(f32),
         mask.reshape(B * L, 1).astype(f32),
         jnp.zeros((B * L, 6), f32)], axis=1)

    xk_knn, zz_knn, pk_flat = _make_sc_gather()(xf, z2_tab, p4, idx_x, idx_z2)
    pk_knn = pk_flat.reshape(N, 4)

    idxf = idx_x.reshape(N, 1)

    w1 = Wout[:, :H * C].T
    w2 = Wout[:, H * C:H * C + H * QK].T
    base3 = H * C + H * QK
    wpts = Wout[:, base3:base3 + 36].reshape(F, H, 3).transpose(2, 1, 0)
    wdist = Wout[:, base3 + 36:base3 + 48].T
    wdir = Wout[:, base3 + 48:base3 + 84].reshape(F, H, 3).transpose(2, 1, 0)
    w3 = jnp.concatenate(
        [wpts.reshape(36, F), wdist, wdir.reshape(36, F),
         jnp.zeros((12, F), f32)], axis=0)

    out = _tc_attention(
        xf, xk_knn, zz_knn, pk_knn, idxf, geo,
        Wq.T, Wk.T, Wv.T, Wpair.T,
        _SEG, _SEGT, _SEG768, _TILE12, _S8,
        w1, w2, w3,
        bout.reshape(1, F), spatial_coef.reshape(1, H),
        ln_gamma.reshape(1, F), ln_beta.reshape(1, F))
    return out.reshape(B, L, F)


# speedup vs baseline: 3.9856x; 1.0278x over previous
import functools

import jax
import jax.numpy as jnp
import numpy as np
from jax import lax
from jax.experimental import pallas as pl
from jax.experimental.pallas import tpu as pltpu
from jax.experimental.pallas import tpu_sc as plsc

B, L, M, F, C, H, QK, VD = 2, 384, 32, 128, 64, 12, 16, 16
N = B * L * M
NC, NS = 2, 16
NW = NC * NS
PER_W = N // NW
CHUNK = 128
NCH = PER_W // CHUNK

TQ = 64
TR = TQ * M
GRID = (B * L) // TQ


def _sc_gather_body(x_hbm, z_hbm, p_hbm, ix_hbm, iz_hbm,
                    xk_out, zz_out, pk_out,
                    ixv, izv, xbuf, zbuf, ptab, pkbuf, sx, sz):
    wid = lax.axis_index("s") * NC + lax.axis_index("c")
    base = wid * PER_W
    ia = pltpu.async_copy(ix_hbm.at[pl.ds(base, PER_W)], ixv, sx.at[0])
    ib = pltpu.async_copy(iz_hbm.at[pl.ds(base, PER_W)], izv, sz.at[0])
    pltpu.sync_copy(p_hbm, ptab)
    ia.wait()
    ib.wait()

    def issue(ci, slot):
        a = pltpu.async_copy(
            x_hbm.at[ixv.at[pl.ds(ci * CHUNK, CHUNK)]], xbuf.at[slot], sx.at[slot])
        b = pltpu.async_copy(
            z_hbm.at[izv.at[pl.ds(ci * CHUNK, CHUNK)]], zbuf.at[slot], sz.at[slot])
        return a, b

    pend = issue(0, 0)
    for ci in range(NCH):
        slot = ci % 2
        off = base + ci * CHUNK
        nxt = issue(ci + 1, 1 - slot) if ci + 1 < NCH else None

        def ploop(i2, c2, ci=ci):
            o = ci * (CHUNK * 4) + i2 * 16 + lax.iota(jnp.int32, 16)
            iv = plsc.load_gather(ixv, [o >> 2])
            vals = plsc.load_gather(ptab, [iv * 4 + (o & 3)])
            pkbuf[pl.ds(i2 * 16, 16)] = vals
            return c2

        lax.fori_loop(0, CHUNK * 4 // 16, ploop, 0)
        pltpu.sync_copy(pkbuf, pk_out.at[pl.ds(off * 4, CHUNK * 4)])
        a, b = pend
        a.wait()
        b.wait()
        pltpu.sync_copy(xbuf.at[slot], xk_out.at[pl.ds(off, CHUNK)])
        pltpu.sync_copy(zbuf.at[slot], zz_out.at[pl.ds(off, CHUNK)])
        pend = nxt


@functools.cache
def _make_sc_gather():
    return pl.kernel(
        _sc_gather_body,
        out_type=[
            jax.ShapeDtypeStruct((N, F), jnp.float32),
            jax.ShapeDtypeStruct((N, 2 * C), jnp.float32),
            jax.ShapeDtypeStruct((N * 4,), jnp.float32),
        ],
        mesh=plsc.VectorSubcoreMesh(
            core_axis_name="c", subcore_axis_name="s",
            num_cores=NC, num_subcores=NS),
        compiler_params=pltpu.CompilerParams(needs_layout_passes=False),
        scratch_types=[
            pltpu.VMEM((PER_W,), jnp.int32),
            pltpu.VMEM((PER_W,), jnp.int32),
            pltpu.VMEM((2, CHUNK, F), jnp.float32),
            pltpu.VMEM((2, CHUNK, 2 * C), jnp.float32),
            pltpu.VMEM((B * L * 4,), jnp.float32),
            pltpu.VMEM((CHUNK * 4,), jnp.float32),
            pltpu.SemaphoreType.DMA((2,)),
            pltpu.SemaphoreType.DMA((2,)),
        ],
    )


def _tc_body(x_ref, xk_ref, zz_ref, pk_ref, idx_ref, geo_ref,
             wq_ref, wk_ref, wv_ref, wp_ref,
             seg_ref, segt_ref, s8_ref,
             w1_ref, w2_ref, w3_ref, bout_ref, coef_ref, lng_ref, lnb_ref,
             o_ref):
    x = x_ref[...]
    xk = xk_ref[...]
    zz = zz_ref[...]
    pk4 = pk_ref[...]
    idxi = idx_ref[...]
    geo = geo_ref[...]

    q = jnp.dot(x, wq_ref[...], preferred_element_type=jnp.float32)
    k = jnp.dot(xk, wk_ref[...], preferred_element_type=jnp.float32)
    v = jnp.dot(xk, wv_ref[...], preferred_element_type=jnp.float32)

    par = (idxi & 1) > 0
    zk2 = jnp.where(par, zz[:, C:], zz[:, :C])

    qb = jnp.broadcast_to(q.reshape(TQ, 1, H * QK),
                          (TQ, M, H * QK)).reshape(TR, H * QK)
    ln2 = jnp.dot(k * qb, seg_ref[...], preferred_element_type=jnp.float32)

    lp2 = jnp.dot(zk2, wp_ref[...], preferred_element_type=jnp.float32)

    pselfb = jnp.broadcast_to(geo[:, 0:4].reshape(TQ, 1, 4),
                              (TQ, M, 4)).reshape(TR, 4)
    diff = pk4 - pselfb
    d2 = jnp.sum(diff * diff, axis=-1, keepdims=True)
    gamma = jnp.log(1.0 + jnp.exp(coef_ref[...]))
    ls2 = d2 * (gamma * (-np.sqrt(2.0 / 9.0) / 2.0))

    logits = (ln2 + lp2 + ls2) * np.sqrt(1.0 / 3.0)
    lg3 = logits.reshape(TQ, M, H)
    mx = jnp.max(lg3, axis=1, keepdims=True)
    e = jnp.exp(lg3 - mx)
    alpha3 = e / jnp.sum(e, axis=1, keepdims=True)
    alpha = alpha3.reshape(TR, H)

    s8 = s8_ref[...]
    av = v * jnp.dot(alpha, segt_ref[...], preferred_element_type=jnp.float32)
    fnode = jnp.dot(s8, av, preferred_element_type=jnp.float32)

    zp = jnp.concatenate([zk2, pk4], axis=1).reshape(TQ, M, C + 4)
    agg = jax.lax.dot_general(alpha.reshape(TQ, M, H), zp,
                              (((1,), (1,)), ((0,), (0,))),
                              preferred_element_type=jnp.float32)
    fp2n = agg[:, :, :C].reshape(TQ, H * C)
    ag = [agg[:, :, C + j].reshape(TQ, H) for j in range(3)]
    am = [ag[j] - geo[:, 4 + j:5 + j] for j in range(3)]
    fpt = [am[0] * geo[:, 8 + i:9 + i] + am[1] * geo[:, 11 + i:12 + i]
           + am[2] * geo[:, 14 + i:15 + i] for i in range(3)]
    dist = jnp.sqrt(fpt[0] * fpt[0] + fpt[1] * fpt[1] + fpt[2] * fpt[2])
    inv = 1.0 / (dist + 1e-4)
    small = jnp.concatenate(
        [fpt[0], fpt[1], fpt[2], dist, fpt[0] * inv, fpt[1] * inv,
         fpt[2] * inv, jnp.zeros((TQ, H), jnp.float32)], axis=-1)

    acc = (jnp.dot(fp2n, w1_ref[...], preferred_element_type=jnp.float32)
           + jnp.dot(fnode, w2_ref[...], preferred_element_type=jnp.float32)
           + jnp.dot(small, w3_ref[...], preferred_element_type=jnp.float32)
           + bout_ref[...])
    feat_all = acc * geo[:, 17:18]

    y = x + feat_all
    mu = jnp.mean(y, axis=-1, keepdims=True)
    yc = y - mu
    var = jnp.mean(yc * yc, axis=-1, keepdims=True)
    o_ref[...] = (yc * lax.rsqrt(var + 1e-5) * lng_ref[...]
                  + lnb_ref[...])


def _row_spec(rows, cols):
    return pl.BlockSpec((rows, cols), lambda i: (i, 0))


def _full_spec(rows, cols):
    return pl.BlockSpec((rows, cols), lambda i: (0, 0))


_TC_IN_SPECS = [
    _row_spec(TQ, F),
    _row_spec(TR, F),
    _row_spec(TR, 2 * C),
    _row_spec(TR, 4),
    _row_spec(TR, 1),
    _row_spec(TQ, 24),
    _full_spec(F, H * QK),
    _full_spec(F, H * QK),
    _full_spec(F, H * VD),
    _full_spec(C, H),
    _full_spec(H * QK, H),
    _full_spec(H, H * QK),
    _full_spec(TQ, TR),
    _full_spec(H * C, F),
    _full_spec(H * QK, F),
    _full_spec(96, F),
    _full_spec(1, F),
    _full_spec(1, H),
    _full_spec(1, F),
    _full_spec(1, F),
]

_tc_attention = pl.pallas_call(
    _tc_body,
    grid=(GRID,),
    in_specs=_TC_IN_SPECS,
    out_specs=_row_spec(TQ, F),
    out_shape=jax.ShapeDtypeStruct((B * L, F), jnp.float32),
)

_SEG = np.kron(np.eye(H), np.ones((QK, 1))).astype(np.float32)
_SEGT = np.kron(np.eye(H), np.ones((1, QK))).astype(np.float32)
_S8 = np.kron(np.eye(TQ), np.ones((1, M))).astype(np.float32)


def kernel(R, t, p_CB, x, z, mask, neighbors, Wq, Wk, Wv, Wpair,
           spatial_coef, Wout, bout, ln_gamma, ln_beta):
    f32 = jnp.float32
    nb = neighbors.astype(jnp.int32)
    boff = (jnp.arange(B, dtype=jnp.int32) * L)[:, None, None]
    idx_x = (nb + boff).reshape(-1)
    qoff = (jnp.arange(B * L, dtype=jnp.int32) * L).reshape(B, L, 1)
    idx_z2 = ((nb + qoff) >> 1).reshape(-1)

    xf = x.reshape(B * L, F).astype(f32)
    z2_tab = z.reshape(B * L * L // 2, 2 * C).astype(f32)
    p4 = jnp.concatenate(
        [p_CB.reshape(B * L, 3).astype(f32), jnp.zeros((B * L, 1), f32)],
        axis=1).reshape(-1)

    geo = jnp.concatenate(
        [p_CB.reshape(B * L, 3).astype(f32), jnp.zeros((B * L, 1), f32),
         t.reshape(B * L, 3).astype(f32), jnp.zeros((B * L, 1), f32),
         R.reshape(B * L, 9).astype(f32),
         mask.reshape(B * L, 1).astype(f32),
         jnp.zeros((B * L, 6), f32)], axis=1)

    xk_knn, zz_knn, pk_flat = _make_sc_gather()(xf, z2_tab, p4, idx_x, idx_z2)
    pk_knn = pk_flat.reshape(N, 4)

    idxf = idx_x.reshape(N, 1)

    w1 = Wout[:, :H * C].T
    w2 = Wout[:, H * C:H * C + H * QK].T
    base3 = H * C + H * QK
    wpts = Wout[:, base3:base3 + 36].reshape(F, H, 3).transpose(2, 1, 0)
    wdist = Wout[:, base3 + 36:base3 + 48].T
    wdir = Wout[:, base3 + 48:base3 + 84].reshape(F, H, 3).transpose(2, 1, 0)
    w3 = jnp.concatenate(
        [wpts.reshape(36, F), wdist, wdir.reshape(36, F),
         jnp.zeros((12, F), f32)], axis=0)

    out = _tc_attention(
        xf, xk_knn, zz_knn, pk_knn, idxf, geo,
        Wq.T, Wk.T, Wv.T, Wpair.T,
        _SEG, _SEGT, _S8,
        w1, w2, w3,
        bout.reshape(1, F), spatial_coef.reshape(1, H),
        ln_gamma.reshape(1, F), ln_beta.reshape(1, F))
    return out.reshape(B, L, F)
